# Initial kernel scaffold; baseline (speedup 1.0000x reference)
import jax
import jax.numpy as jnp
from jax.experimental import pallas as pl


def kernel(InState, GnnPerms, NNsites, SitesToShells, gdiags, Psi0, bias0, Psi1, bias1, Psi2, bias2, Psi3, bias3, Psi4, bias4, PsiR3):
    raise NotImplementedError("write your pallas kernel here")


# speedup vs baseline: 28.3626x; 28.3626x over previous
import functools

import jax
import jax.numpy as jnp
from jax import lax
from jax.experimental import pallas as pl
from jax.experimental.pallas import tpu as pltpu
from jax.experimental.pallas import tpu_sc as plsc

NB = 2
NSPEC = 2
NSITES = 50000
N_NGB = 16
NG = 48
NCH = 4
NSHELLS = 8
DIM = 3

NPAD = 51200
NROWS = NPAD * N_NGB
NW = 32
PER_TILE = NROWS // NW
CHUNK = 128
NCHUNK = PER_TILE // CHUNK
GRP = 8
BLK = 2048
NSTEPS = NPAD // BLK


def _sc_gather_body(idx_hbm, table_hbm, out_hbm, idx_v, buf_v, gsem):
    c = lax.axis_index("c")
    s = lax.axis_index("s")
    wid = s * 2 + c
    base = wid * PER_TILE
    pltpu.sync_copy(idx_hbm.at[wid], idx_v)

    def round_(jo, carry):
        copies = []
        for b in range(GRP):
            j = jo * GRP + b
            copies.append(pltpu.async_copy(
                table_hbm.at[idx_v.at[j]],
                buf_v.at[pl.ds(b * CHUNK, CHUNK)],
                gsem))
        for cp in copies:
            cp.wait()
        pltpu.sync_copy(buf_v, out_hbm.at[pl.ds(base + jo * (GRP * CHUNK),
                                                GRP * CHUNK)])
        return carry

    lax.fori_loop(0, NCHUNK // GRP, round_, 0)


@functools.cache
def _sc_gather():
    return pl.kernel(
        _sc_gather_body,
        out_type=jax.ShapeDtypeStruct((NROWS, 8), jnp.float32),
        mesh=plsc.VectorSubcoreMesh(core_axis_name="c", subcore_axis_name="s"),
        scratch_types=[
            pltpu.VMEM((NCHUNK, CHUNK), jnp.int32),
            pltpu.VMEM((GRP * CHUNK, 8), jnp.float32),
            pltpu.SemaphoreType.DMA,
        ],
        compiler_params=pltpu.CompilerParams(use_tc_tiling_on_sc=False),
    )


def _softplus(h):
    return jnp.maximum(h, 0.0) + jnp.log1p(jnp.exp(-jnp.abs(h)))


def _conv_body(xn_ref, w_ref, b_ref, s_ref, y_ref):
    h = jnp.dot(xn_ref[...], w_ref[...], preferred_element_type=jnp.float32)
    h = _softplus(h + b_ref[...])
    y_ref[...] = jnp.dot(h, s_ref[...], preferred_element_type=jnp.float32)


def _conv(xn2, w, bv, s):
    ncol = w.shape[1]
    return pl.pallas_call(
        _conv_body,
        grid=(NSTEPS,),
        in_specs=[
            pl.BlockSpec((BLK, 128), lambda i: (i, 0)),
            pl.BlockSpec((128, ncol), lambda i: (0, 0)),
            pl.BlockSpec((1, ncol), lambda i: (0, 0)),
            pl.BlockSpec((ncol, 8), lambda i: (0, 0)),
        ],
        out_specs=pl.BlockSpec((BLK, 8), lambda i: (i, 0)),
        out_shape=jax.ShapeDtypeStruct((NPAD, 8), jnp.float32),
    )(xn2, w, bv, s)


def _head_body(xn_ref, w_ref, b_ref, s_ref, sts_ref, svg_ref, out_ref):
    h = jnp.dot(xn_ref[...], w_ref[...], preferred_element_type=jnp.float32)
    h = _softplus(h + b_ref[...])
    y5p = jnp.dot(h, s_ref[...], preferred_element_type=jnp.float32)
    oh = (sts_ref[...] == lax.broadcasted_iota(jnp.int32, (BLK, 8), 1))
    svp = jnp.dot(oh.astype(jnp.float32), svg_ref[...],
                  preferred_element_type=jnp.float32)
    part = lax.dot_general(y5p, svp, (((0,), (0,)), ((), ())),
                           preferred_element_type=jnp.float32)

    @pl.when(pl.program_id(0) == 0)
    def _():
        out_ref[...] = jnp.zeros_like(out_ref)

    out_ref[...] += part


def _head(xn2, w, bv, s4, sts2d, svgp):
    ncol = w.shape[1]
    return pl.pallas_call(
        _head_body,
        grid=(NSTEPS,),
        in_specs=[
            pl.BlockSpec((BLK, 128), lambda i: (i, 0)),
            pl.BlockSpec((128, ncol), lambda i: (0, 0)),
            pl.BlockSpec((1, ncol), lambda i: (0, 0)),
            pl.BlockSpec((ncol, 8), lambda i: (0, 0)),
            pl.BlockSpec((BLK, 1), lambda i: (i, 0)),
            pl.BlockSpec((8, 128), lambda i: (0, 0)),
        ],
        out_specs=pl.BlockSpec((8, 128), lambda i: (0, 0)),
        out_shape=jax.ShapeDtypeStruct((8, 128), jnp.float32),
    )(xn2, w, bv, s4, sts2d, svgp)


def _build_w(Psi, GnnPerms):
    Cout, Cin, _ = Psi.shape
    GP = jnp.take(Psi, GnnPerms, axis=2)
    T = jnp.transpose(GP, (3, 1, 2, 0))
    Tp = jnp.zeros((N_NGB, NCH, NG, Cout), jnp.float32).at[:, :Cin].set(T)
    W = jnp.einsum('jcgo,bB->jbcBgo', Tp, jnp.eye(NB, dtype=jnp.float32))
    return W.reshape(N_NGB * NB * NCH, NB * NG * Cout)


def kernel(InState, GnnPerms, NNsites, SitesToShells, gdiags,
           Psi0, bias0, Psi1, bias1, Psi2, bias2, Psi3, bias3, Psi4, bias4,
           PsiR3):
    f32 = jnp.float32

    x = jnp.zeros((NPAD, 8), f32)
    x = x.at[:NSITES, 0:NSPEC].set(InState[0].T)
    x = x.at[:NSITES, NCH:NCH + NSPEC].set(InState[1].T)

    idxT = jnp.zeros((NPAD, N_NGB), jnp.int32).at[:NSITES].set(NNsites.T)
    idx = idxT.reshape(NW, NCHUNK, CHUNK)

    Ws = [_build_w(P, GnnPerms) for P in (Psi0, Psi1, Psi2, Psi3, Psi4)]
    bvs = [jnp.tile(b, NB * NG)[None, :] for b in
           (bias0, bias1, bias2, bias3, bias4)]
    S = jnp.kron(jnp.eye(NB, dtype=f32),
                 jnp.tile(jnp.eye(NCH, dtype=f32), (NG, 1)) / NG)
    S4 = jnp.kron(jnp.eye(NB, dtype=f32), jnp.ones((NG, 1), f32) / NG)
    S4p = jnp.zeros((NB * NG, 8), f32).at[:, :NB].set(S4)

    Gmean = jnp.mean(gdiags, axis=0)
    svG = PsiR3 @ Gmean.T
    svGp = jnp.zeros((NSHELLS, 128), f32).at[:, :DIM].set(svG)
    sts2d = jnp.full((NPAD, 1), NSHELLS, jnp.int32
                     ).at[:NSITES, 0].set(SitesToShells)

    gather = _sc_gather()
    for l in range(4):
        xn = gather(idx, x).reshape(NPAD, 128)
        x = _conv(xn, Ws[l], bvs[l], S)
    xn = gather(idx, x).reshape(NPAD, 128)
    out = _head(xn, Ws[4], bvs[4], S4p, sts2d, svGp)
    return out[:NB, :DIM]


# speedup vs baseline: 29.7636x; 1.0494x over previous
import functools

import jax
import jax.numpy as jnp
from jax import lax
from jax.experimental import pallas as pl
from jax.experimental.pallas import tpu as pltpu
from jax.experimental.pallas import tpu_sc as plsc

NB = 2
NSPEC = 2
NSITES = 50000
N_NGB = 16
NG = 48
NCH = 4
NSHELLS = 8
DIM = 3

NPAD = 51200
NROWS = NPAD * N_NGB
NW = 32
PER_TILE = NROWS // NW
CHUNK = 128
NCHUNK = PER_TILE // CHUNK
GRP = 5
NROUND = NCHUNK // GRP
RSZ = GRP * CHUNK
NSLOT = 4
BLK = 2048
NSTEPS = NPAD // BLK


def _sc_gather_body(idx_hbm, table_hbm, out_hbm, idx_v, buf_v, gsem, wsem):
    c = lax.axis_index("c")
    s = lax.axis_index("s")
    wid = s * 2 + c
    base = wid * PER_TILE
    pltpu.sync_copy(idx_hbm.at[wid], idx_v)

    def fire(r, slot):
        for k in range(GRP):
            pltpu.async_copy(table_hbm.at[idx_v.at[r * GRP + k]],
                             buf_v.at[slot, pl.ds(k * CHUNK, CHUNK)], gsem)

    def drain(slot):
        for k in range(GRP):
            pltpu.make_async_copy(
                table_hbm.at[idx_v.at[0]],
                buf_v.at[slot, pl.ds(k * CHUNK, CHUNK)], gsem).wait()

    def write(r, slot):
        pltpu.async_copy(buf_v.at[slot], out_hbm.at[pl.ds(base + r * RSZ, RSZ)],
                         wsem)

    def wait_write(slot):
        pltpu.make_async_copy(buf_v.at[slot],
                              out_hbm.at[pl.ds(base, RSZ)], wsem).wait()

    fire(0, 0)

    def body(j, carry):
        for b in range(NSLOT):
            r = NSLOT * j + b
            nslot = (b + 1) % NSLOT
            if b < NSLOT - 1:
                @pl.when(j > 0)
                def _():
                    wait_write(nslot)
            else:
                wait_write(0)

            @pl.when(r + 1 < NROUND)
            def _():
                fire(r + 1, nslot)
            drain(b)
            write(r, b)
        return carry

    lax.fori_loop(0, NROUND // NSLOT, body, 0)
    for slot in range(1, NSLOT):
        wait_write(slot)


@functools.cache
def _sc_gather():
    return pl.kernel(
        _sc_gather_body,
        out_type=jax.ShapeDtypeStruct((NROWS, 8), jnp.float32),
        mesh=plsc.VectorSubcoreMesh(core_axis_name="c", subcore_axis_name="s"),
        scratch_types=[
            pltpu.VMEM((NCHUNK, CHUNK), jnp.int32),
            pltpu.VMEM((NSLOT, RSZ, 8), jnp.float32),
            pltpu.SemaphoreType.DMA,
            pltpu.SemaphoreType.DMA,
        ],
        compiler_params=pltpu.CompilerParams(use_tc_tiling_on_sc=False),
    )


def _softplus(h):
    return jnp.maximum(h, 0.0) + jnp.log1p(jnp.exp(-jnp.abs(h)))


def _conv_body(xn_ref, w_ref, b_ref, s_ref, y_ref):
    h = jnp.dot(xn_ref[...], w_ref[...], preferred_element_type=jnp.float32)
    h = _softplus(h + b_ref[...])
    y_ref[...] = jnp.dot(h, s_ref[...], preferred_element_type=jnp.float32)


def _conv(xn2, w, bv, s):
    ncol = w.shape[1]
    return pl.pallas_call(
        _conv_body,
        grid=(NSTEPS,),
        in_specs=[
            pl.BlockSpec((BLK, 128), lambda i: (i, 0)),
            pl.BlockSpec((128, ncol), lambda i: (0, 0)),
            pl.BlockSpec((1, ncol), lambda i: (0, 0)),
            pl.BlockSpec((ncol, 8), lambda i: (0, 0)),
        ],
        out_specs=pl.BlockSpec((BLK, 8), lambda i: (i, 0)),
        out_shape=jax.ShapeDtypeStruct((NPAD, 8), jnp.float32),
    )(xn2, w, bv, s)


def _head_body(xn_ref, w_ref, b_ref, s_ref, sts_ref, svg_ref, out_ref):
    h = jnp.dot(xn_ref[...], w_ref[...], preferred_element_type=jnp.float32)
    h = _softplus(h + b_ref[...])
    y5p = jnp.dot(h, s_ref[...], preferred_element_type=jnp.float32)
    oh = (sts_ref[...] == lax.broadcasted_iota(jnp.int32, (BLK, 8), 1))
    svp = jnp.dot(oh.astype(jnp.float32), svg_ref[...],
                  preferred_element_type=jnp.float32)
    part = lax.dot_general(y5p, svp, (((0,), (0,)), ((), ())),
                           preferred_element_type=jnp.float32)

    @pl.when(pl.program_id(0) == 0)
    def _():
        out_ref[...] = jnp.zeros_like(out_ref)

    out_ref[...] += part


def _head(xn2, w, bv, s4, sts2d, svgp):
    ncol = w.shape[1]
    return pl.pallas_call(
        _head_body,
        grid=(NSTEPS,),
        in_specs=[
            pl.BlockSpec((BLK, 128), lambda i: (i, 0)),
            pl.BlockSpec((128, ncol), lambda i: (0, 0)),
            pl.BlockSpec((1, ncol), lambda i: (0, 0)),
            pl.BlockSpec((ncol, 8), lambda i: (0, 0)),
            pl.BlockSpec((BLK, 1), lambda i: (i, 0)),
            pl.BlockSpec((8, 128), lambda i: (0, 0)),
        ],
        out_specs=pl.BlockSpec((8, 128), lambda i: (0, 0)),
        out_shape=jax.ShapeDtypeStruct((8, 128), jnp.float32),
    )(xn2, w, bv, s4, sts2d, svgp)


def _build_w(Psi, GnnPerms):
    Cout, Cin, _ = Psi.shape
    GP = jnp.take(Psi, GnnPerms, axis=2)
    T = jnp.transpose(GP, (3, 1, 2, 0))
    Tp = jnp.zeros((N_NGB, NCH, NG, Cout), jnp.float32).at[:, :Cin].set(T)
    W = jnp.einsum('jcgo,bB->jbcBgo', Tp, jnp.eye(NB, dtype=jnp.float32))
    return W.reshape(N_NGB * NB * NCH, NB * NG * Cout)


def kernel(InState, GnnPerms, NNsites, SitesToShells, gdiags,
           Psi0, bias0, Psi1, bias1, Psi2, bias2, Psi3, bias3, Psi4, bias4,
           PsiR3):
    f32 = jnp.float32

    x = jnp.zeros((NPAD, 8), f32)
    x = x.at[:NSITES, 0:NSPEC].set(InState[0].T)
    x = x.at[:NSITES, NCH:NCH + NSPEC].set(InState[1].T)

    idxT = jnp.zeros((NPAD, N_NGB), jnp.int32).at[:NSITES].set(NNsites.T)
    idx = idxT.reshape(NW, NCHUNK, CHUNK)

    Ws = [_build_w(P, GnnPerms) for P in (Psi0, Psi1, Psi2, Psi3, Psi4)]
    bvs = [jnp.tile(b, NB * NG)[None, :] for b in
           (bias0, bias1, bias2, bias3, bias4)]
    S = jnp.kron(jnp.eye(NB, dtype=f32),
                 jnp.tile(jnp.eye(NCH, dtype=f32), (NG, 1)) / NG)
    S4 = jnp.kron(jnp.eye(NB, dtype=f32), jnp.ones((NG, 1), f32) / NG)
    S4p = jnp.zeros((NB * NG, 8), f32).at[:, :NB].set(S4)

    Gmean = jnp.mean(gdiags, axis=0)
    svG = PsiR3 @ Gmean.T
    svGp = jnp.zeros((NSHELLS, 128), f32).at[:, :DIM].set(svG)
    sts2d = jnp.full((NPAD, 1), NSHELLS, jnp.int32
                     ).at[:NSITES, 0].set(SitesToShells)

    gather = _sc_gather()
    for l in range(4):
        xn = gather(idx, x).reshape(NPAD, 128)
        x = _conv(xn, Ws[l], bvs[l], S)
    xn = gather(idx, x).reshape(NPAD, 128)
    out = _head(xn, Ws[4], bvs[4], S4p, sts2d, svGp)
    return out[:NB, :DIM]


# speedup vs baseline: 47.5109x; 1.5963x over previous
import functools

import jax
import jax.numpy as jnp
from jax import lax
from jax.experimental import pallas as pl
from jax.experimental.pallas import tpu as pltpu
from jax.experimental.pallas import tpu_sc as plsc

NB = 2
NSPEC = 2
NSITES = 50000
N_NGB = 16
NG = 48
NCH = 4
NSHELLS = 8
DIM = 3

NPAD = 51200
NROWS = NPAD * N_NGB
NW = 32
PER_TILE = NROWS // NW
CHUNK = 128
NCHUNK = PER_TILE // CHUNK
GRP = 5
NROUND = NCHUNK // GRP
RSZ = GRP * CHUNK
NSLOT = 4
BLK = 2048
NSTEPS = NPAD // BLK


def _sc_gather_body(idx_hbm, table_hbm, out_hbm, idx_v, buf_v, shared, gsem,
                    wsem):
    c = lax.axis_index("c")
    s = lax.axis_index("s")
    wid = s * 2 + c
    base = wid * PER_TILE

    @pl.when(s == 0)
    def _():
        pltpu.sync_copy(table_hbm, shared)
    pltpu.sync_copy(idx_hbm.at[wid], idx_v)
    plsc.subcore_barrier()

    def fire(r, slot):
        for k in range(GRP):
            pltpu.async_copy(shared.at[idx_v.at[r * GRP + k]],
                             buf_v.at[slot, pl.ds(k * CHUNK, CHUNK)], gsem)

    def drain(slot):
        for k in range(GRP):
            pltpu.make_async_copy(
                shared.at[idx_v.at[0]],
                buf_v.at[slot, pl.ds(k * CHUNK, CHUNK)], gsem).wait()

    def write(r, slot):
        pltpu.async_copy(buf_v.at[slot], out_hbm.at[pl.ds(base + r * RSZ, RSZ)],
                         wsem)

    def wait_write(slot):
        pltpu.make_async_copy(buf_v.at[slot],
                              out_hbm.at[pl.ds(base, RSZ)], wsem).wait()

    fire(0, 0)

    def body(j, carry):
        for b in range(NSLOT):
            r = NSLOT * j + b
            nslot = (b + 1) % NSLOT
            if b < NSLOT - 1:
                @pl.when(j > 0)
                def _():
                    wait_write(nslot)
            else:
                wait_write(0)

            @pl.when(r + 1 < NROUND)
            def _():
                fire(r + 1, nslot)
            drain(b)
            write(r, b)
        return carry

    lax.fori_loop(0, NROUND // NSLOT, body, 0)
    for slot in range(1, NSLOT):
        wait_write(slot)


@functools.cache
def _sc_gather():
    return pl.kernel(
        _sc_gather_body,
        out_type=jax.ShapeDtypeStruct((NROWS, 8), jnp.float32),
        mesh=plsc.VectorSubcoreMesh(core_axis_name="c", subcore_axis_name="s"),
        scratch_types=[
            pltpu.VMEM((NCHUNK, CHUNK), jnp.int32),
            pltpu.VMEM((NSLOT, RSZ, 8), jnp.float32),
            pltpu.VMEM_SHARED((NPAD, 8), jnp.float32),
            pltpu.SemaphoreType.DMA,
            pltpu.SemaphoreType.DMA,
        ],
        compiler_params=pltpu.CompilerParams(use_tc_tiling_on_sc=False),
    )


def _softplus(h):
    return jnp.maximum(h, 0.0) + jnp.log1p(jnp.exp(-jnp.abs(h)))


def _conv_body(xn_ref, w_ref, b_ref, s_ref, y_ref):
    h = jnp.dot(xn_ref[...], w_ref[...], preferred_element_type=jnp.float32)
    h = _softplus(h + b_ref[...])
    y_ref[...] = jnp.dot(h, s_ref[...], preferred_element_type=jnp.float32)


def _conv(xn2, w, bv, s):
    ncol = w.shape[1]
    return pl.pallas_call(
        _conv_body,
        grid=(NSTEPS,),
        in_specs=[
            pl.BlockSpec((BLK, 128), lambda i: (i, 0)),
            pl.BlockSpec((128, ncol), lambda i: (0, 0)),
            pl.BlockSpec((1, ncol), lambda i: (0, 0)),
            pl.BlockSpec((ncol, 8), lambda i: (0, 0)),
        ],
        out_specs=pl.BlockSpec((BLK, 8), lambda i: (i, 0)),
        out_shape=jax.ShapeDtypeStruct((NPAD, 8), jnp.float32),
    )(xn2, w, bv, s)


def _head_body(xn_ref, w_ref, b_ref, s_ref, sts_ref, svg_ref, out_ref):
    h = jnp.dot(xn_ref[...], w_ref[...], preferred_element_type=jnp.float32)
    h = _softplus(h + b_ref[...])
    y5p = jnp.dot(h, s_ref[...], preferred_element_type=jnp.float32)
    oh = (sts_ref[...] == lax.broadcasted_iota(jnp.int32, (BLK, 8), 1))
    svp = jnp.dot(oh.astype(jnp.float32), svg_ref[...],
                  preferred_element_type=jnp.float32)
    part = lax.dot_general(y5p, svp, (((0,), (0,)), ((), ())),
                           preferred_element_type=jnp.float32)

    @pl.when(pl.program_id(0) == 0)
    def _():
        out_ref[...] = jnp.zeros_like(out_ref)

    out_ref[...] += part


def _head(xn2, w, bv, s4, sts2d, svgp):
    ncol = w.shape[1]
    return pl.pallas_call(
        _head_body,
        grid=(NSTEPS,),
        in_specs=[
            pl.BlockSpec((BLK, 128), lambda i: (i, 0)),
            pl.BlockSpec((128, ncol), lambda i: (0, 0)),
            pl.BlockSpec((1, ncol), lambda i: (0, 0)),
            pl.BlockSpec((ncol, 8), lambda i: (0, 0)),
            pl.BlockSpec((BLK, 1), lambda i: (i, 0)),
            pl.BlockSpec((8, 128), lambda i: (0, 0)),
        ],
        out_specs=pl.BlockSpec((8, 128), lambda i: (0, 0)),
        out_shape=jax.ShapeDtypeStruct((8, 128), jnp.float32),
    )(xn2, w, bv, s4, sts2d, svgp)


def _build_w(Psi, GnnPerms):
    Cout, Cin, _ = Psi.shape
    GP = jnp.take(Psi, GnnPerms, axis=2)
    T = jnp.transpose(GP, (3, 1, 2, 0))
    Tp = jnp.zeros((N_NGB, NCH, NG, Cout), jnp.float32).at[:, :Cin].set(T)
    W = jnp.einsum('jcgo,bB->jbcBgo', Tp, jnp.eye(NB, dtype=jnp.float32))
    return W.reshape(N_NGB * NB * NCH, NB * NG * Cout)


def kernel(InState, GnnPerms, NNsites, SitesToShells, gdiags,
           Psi0, bias0, Psi1, bias1, Psi2, bias2, Psi3, bias3, Psi4, bias4,
           PsiR3):
    f32 = jnp.float32

    x = jnp.zeros((NPAD, 8), f32)
    x = x.at[:NSITES, 0:NSPEC].set(InState[0].T)
    x = x.at[:NSITES, NCH:NCH + NSPEC].set(InState[1].T)

    idxT = jnp.zeros((NPAD, N_NGB), jnp.int32).at[:NSITES].set(NNsites.T)
    idx = idxT.reshape(NW, NCHUNK, CHUNK)

    Ws = [_build_w(P, GnnPerms) for P in (Psi0, Psi1, Psi2, Psi3, Psi4)]
    bvs = [jnp.tile(b, NB * NG)[None, :] for b in
           (bias0, bias1, bias2, bias3, bias4)]
    S = jnp.kron(jnp.eye(NB, dtype=f32),
                 jnp.tile(jnp.eye(NCH, dtype=f32), (NG, 1)) / NG)
    S4 = jnp.kron(jnp.eye(NB, dtype=f32), jnp.ones((NG, 1), f32) / NG)
    S4p = jnp.zeros((NB * NG, 8), f32).at[:, :NB].set(S4)

    svGp = jnp.zeros((NSHELLS, 128), f32).at[:, :DIM].set(PsiR3)
    sts2d = jnp.full((NPAD, 1), NSHELLS, jnp.int32
                     ).at[:NSITES, 0].set(SitesToShells)

    gather = _sc_gather()
    for l in range(4):
        xn = gather(idx, x).reshape(NPAD, 128)
        x = _conv(xn, Ws[l], bvs[l], S)
    xn = gather(idx, x).reshape(NPAD, 128)
    a = _head(xn, Ws[4], bvs[4], S4p, sts2d, svGp)[:NB, :DIM]
    return jnp.mean(jnp.einsum('gde,be->bgd', gdiags, a), axis=1)


# speedup vs baseline: 51.5724x; 1.0855x over previous
import functools

import jax
import jax.numpy as jnp
from jax import lax
from jax.experimental import pallas as pl
from jax.experimental.pallas import tpu as pltpu
from jax.experimental.pallas import tpu_sc as plsc

NB = 2
NSPEC = 2
NSITES = 50000
N_NGB = 16
NG = 48
NCH = 4
NSHELLS = 8
DIM = 3

NPAD = 51200
NROWS = NPAD * N_NGB
NW = 32
PER_TILE = NROWS // NW
CHUNK = 128
NCHUNK = PER_TILE // CHUNK
GRP = 5
NROUND = NCHUNK // GRP
RSZ = GRP * CHUNK
NSLOT = 4
BLK = 2048
NSTEPS = NPAD // BLK


def _sc_gather_body(idx_hbm, table_hbm, out_hbm, idx_v, buf_v, shared, gsem,
                    wsem):
    c = lax.axis_index("c")
    s = lax.axis_index("s")
    wid = s * 2 + c
    base = wid * PER_TILE

    srows = NPAD // 16
    pltpu.sync_copy(table_hbm.at[pl.ds(s * srows, srows)],
                    shared.at[pl.ds(s * srows, srows)])
    pltpu.sync_copy(idx_hbm.at[wid], idx_v)
    plsc.subcore_barrier()

    def fire(r, slot):
        for k in range(GRP):
            pltpu.async_copy(shared.at[idx_v.at[r * GRP + k]],
                             buf_v.at[slot, pl.ds(k * CHUNK, CHUNK)], gsem)

    def drain(slot):
        for k in range(GRP):
            pltpu.make_async_copy(
                shared.at[idx_v.at[0]],
                buf_v.at[slot, pl.ds(k * CHUNK, CHUNK)], gsem).wait()

    def write(r, slot):
        pltpu.async_copy(buf_v.at[slot], out_hbm.at[pl.ds(base + r * RSZ, RSZ)],
                         wsem)

    def wait_write(slot):
        pltpu.make_async_copy(buf_v.at[slot],
                              out_hbm.at[pl.ds(base, RSZ)], wsem).wait()

    fire(0, 0)

    def body(j, carry):
        for b in range(NSLOT):
            r = NSLOT * j + b
            nslot = (b + 1) % NSLOT
            if b < NSLOT - 1:
                @pl.when(j > 0)
                def _():
                    wait_write(nslot)
            else:
                wait_write(0)

            @pl.when(r + 1 < NROUND)
            def _():
                fire(r + 1, nslot)
            drain(b)
            write(r, b)
        return carry

    lax.fori_loop(0, NROUND // NSLOT, body, 0)
    for slot in range(1, NSLOT):
        wait_write(slot)


@functools.cache
def _sc_gather():
    return pl.kernel(
        _sc_gather_body,
        out_type=jax.ShapeDtypeStruct((NROWS, 8), jnp.float32),
        mesh=plsc.VectorSubcoreMesh(core_axis_name="c", subcore_axis_name="s"),
        scratch_types=[
            pltpu.VMEM((NCHUNK, CHUNK), jnp.int32),
            pltpu.VMEM((NSLOT, RSZ, 8), jnp.float32),
            pltpu.VMEM_SHARED((NPAD, 8), jnp.float32),
            pltpu.SemaphoreType.DMA,
            pltpu.SemaphoreType.DMA,
        ],
        compiler_params=pltpu.CompilerParams(use_tc_tiling_on_sc=False),
    )


def _softplus(h):
    z = jnp.exp2(jnp.minimum(h, -h) * 1.4426950408889634)
    return jnp.maximum(h, 0.0) + 0.6931471805599453 * jnp.log2(1.0 + z)


def _conv_body(xn_ref, w_ref, b_ref, s_ref, y_ref):
    h = jnp.dot(xn_ref[...], w_ref[...], preferred_element_type=jnp.float32)
    h = _softplus(h + b_ref[...])
    y_ref[...] = jnp.dot(h, s_ref[...], preferred_element_type=jnp.float32)


def _conv(xn2, w, bv, s):
    ncol = w.shape[1]
    return pl.pallas_call(
        _conv_body,
        grid=(NSTEPS,),
        in_specs=[
            pl.BlockSpec((BLK, 128), lambda i: (i, 0)),
            pl.BlockSpec((128, ncol), lambda i: (0, 0)),
            pl.BlockSpec((1, ncol), lambda i: (0, 0)),
            pl.BlockSpec((ncol, 8), lambda i: (0, 0)),
        ],
        out_specs=pl.BlockSpec((BLK, 8), lambda i: (i, 0)),
        out_shape=jax.ShapeDtypeStruct((NPAD, 8), jnp.float32),
    )(xn2, w, bv, s)


def _head_body(xn_ref, w_ref, b_ref, s_ref, sts_ref, svg_ref, out_ref):
    h = jnp.dot(xn_ref[...], w_ref[...], preferred_element_type=jnp.float32)
    h = _softplus(h + b_ref[...])
    y5p = jnp.dot(h, s_ref[...], preferred_element_type=jnp.float32)
    oh = (sts_ref[...] == lax.broadcasted_iota(jnp.int32, (BLK, 8), 1))
    svp = jnp.dot(oh.astype(jnp.float32), svg_ref[...],
                  preferred_element_type=jnp.float32)
    part = lax.dot_general(y5p, svp, (((0,), (0,)), ((), ())),
                           preferred_element_type=jnp.float32)

    @pl.when(pl.program_id(0) == 0)
    def _():
        out_ref[...] = jnp.zeros_like(out_ref)

    out_ref[...] += part


def _head(xn2, w, bv, s4, sts2d, svgp):
    ncol = w.shape[1]
    return pl.pallas_call(
        _head_body,
        grid=(NSTEPS,),
        in_specs=[
            pl.BlockSpec((BLK, 128), lambda i: (i, 0)),
            pl.BlockSpec((128, ncol), lambda i: (0, 0)),
            pl.BlockSpec((1, ncol), lambda i: (0, 0)),
            pl.BlockSpec((ncol, 8), lambda i: (0, 0)),
            pl.BlockSpec((BLK, 1), lambda i: (i, 0)),
            pl.BlockSpec((8, 128), lambda i: (0, 0)),
        ],
        out_specs=pl.BlockSpec((8, 128), lambda i: (0, 0)),
        out_shape=jax.ShapeDtypeStruct((8, 128), jnp.float32),
    )(xn2, w, bv, s4, sts2d, svgp)


def _build_w(Psi, GnnPerms):
    Cout, Cin, _ = Psi.shape
    GP = jnp.take(Psi, GnnPerms, axis=2)
    T = jnp.transpose(GP, (3, 1, 2, 0))
    Tp = jnp.zeros((N_NGB, NCH, NG, Cout), jnp.float32).at[:, :Cin].set(T)
    W = jnp.einsum('jcgo,bB->jbcBgo', Tp, jnp.eye(NB, dtype=jnp.float32))
    return W.reshape(N_NGB * NB * NCH, NB * NG * Cout)


def kernel(InState, GnnPerms, NNsites, SitesToShells, gdiags,
           Psi0, bias0, Psi1, bias1, Psi2, bias2, Psi3, bias3, Psi4, bias4,
           PsiR3):
    f32 = jnp.float32

    x = jnp.zeros((NPAD, 8), f32)
    x = x.at[:NSITES, 0:NSPEC].set(InState[0].T)
    x = x.at[:NSITES, NCH:NCH + NSPEC].set(InState[1].T)

    idxT = jnp.zeros((NPAD, N_NGB), jnp.int32).at[:NSITES].set(NNsites.T)
    idx = idxT.reshape(NW, NCHUNK, CHUNK)

    Ws = [_build_w(P, GnnPerms) for P in (Psi0, Psi1, Psi2, Psi3, Psi4)]
    bvs = [jnp.tile(b, NB * NG)[None, :] for b in
           (bias0, bias1, bias2, bias3, bias4)]
    S = jnp.kron(jnp.eye(NB, dtype=f32),
                 jnp.tile(jnp.eye(NCH, dtype=f32), (NG, 1)) / NG)
    S4 = jnp.kron(jnp.eye(NB, dtype=f32), jnp.ones((NG, 1), f32) / NG)
    S4p = jnp.zeros((NB * NG, 8), f32).at[:, :NB].set(S4)

    svGp = jnp.zeros((NSHELLS, 128), f32).at[:, :DIM].set(PsiR3)
    sts2d = jnp.full((NPAD, 1), NSHELLS, jnp.int32
                     ).at[:NSITES, 0].set(SitesToShells)

    gather = _sc_gather()
    for l in range(4):
        xn = gather(idx, x).reshape(NPAD, 128)
        x = _conv(xn, Ws[l], bvs[l], S)
    xn = gather(idx, x).reshape(NPAD, 128)
    a = _head(xn, Ws[4], bvs[4], S4p, sts2d, svGp)[:NB, :DIM]
    return jnp.mean(jnp.einsum('gde,be->bgd', gdiags, a), axis=1)
